# baseline (speedup 1.0000x reference)
import jax
import jax.numpy as jnp
from jax.experimental import pallas as pl

B, N, T, P, D = 256, 128, 1, 7, 256
TOKEN_DIM = 128
EMB_FEAT = P * D
M = B * N

BB = 4
ROWS = BB * N


def _fused_masked_linear(x_ref, vis_ref, mask_ref, w1_ref, w2_ref, b_ref, out_ref):
    vis2d = vis_ref[:].reshape(ROWS, P)
    acc = jnp.dot(vis2d, w2_ref[:], preferred_element_type=jnp.float32)
    acc += b_ref[:]
    for p in range(P):
        x = x_ref[:, :, 0, p, :].reshape(ROWS, D).astype(jnp.bfloat16)
        acc += jnp.dot(x, w1_ref[p], preferred_element_type=jnp.float32)
    out_ref[:] = acc * mask_ref[:].reshape(ROWS, 1)


def kernel(embs, vis, masks, W, b):
    maskf = masks.astype(jnp.float32)
    w1 = W[:, :EMB_FEAT].T.reshape(P, D, TOKEN_DIM).astype(jnp.bfloat16)
    w2 = W[:, EMB_FEAT:].T
    b2 = b.reshape(1, TOKEN_DIM)

    grid = (B // BB,)
    out = pl.pallas_call(
        _fused_masked_linear,
        grid=grid,
        in_specs=[
            pl.BlockSpec((BB, N, 1, P, D), lambda i: (i, 0, 0, 0, 0)),
            pl.BlockSpec((BB, N, 1, P), lambda i: (i, 0, 0, 0)),
            pl.BlockSpec((BB, N, 1), lambda i: (i, 0, 0)),
            pl.BlockSpec((P, D, TOKEN_DIM), lambda i: (0, 0, 0)),
            pl.BlockSpec((P, TOKEN_DIM), lambda i: (0, 0)),
            pl.BlockSpec((1, TOKEN_DIM), lambda i: (0, 0)),
        ],
        out_specs=pl.BlockSpec((ROWS, TOKEN_DIM), lambda i: (i, 0)),
        out_shape=jax.ShapeDtypeStruct((M, TOKEN_DIM), jnp.float32),
    )(embs, vis, maskf, w1, w2, b2)
    return out.reshape(B, N, TOKEN_DIM)


# speedup vs baseline: 1.3954x; 1.3954x over previous
import jax
import jax.numpy as jnp
from jax.experimental import pallas as pl

B, N, T, P, D = 256, 128, 1, 7, 256
TOKEN_DIM = 128
EMB_FEAT = P * D
M = B * N

BB = 4
ROWS = BB * N


def _fused_masked_linear(x_ref, vis_ref, mask_ref, w1_ref, w2_ref, b_ref, out_ref):
    vis2d = vis_ref[:].reshape(ROWS, P)
    acc = jnp.dot(vis2d, w2_ref[:], preferred_element_type=jnp.float32)
    acc += b_ref[:]
    x4 = x_ref[:, :, 0, :, :]
    xt = jnp.transpose(x4, (2, 0, 1, 3)).astype(jnp.bfloat16)
    for p in range(P):
        acc += jnp.dot(xt[p].reshape(ROWS, D), w1_ref[p],
                       preferred_element_type=jnp.float32)
    out_ref[:] = acc * mask_ref[:].reshape(ROWS, 1)


def kernel(embs, vis, masks, W, b):
    maskf = masks.astype(jnp.float32)
    w1 = W[:, :EMB_FEAT].T.reshape(P, D, TOKEN_DIM).astype(jnp.bfloat16)
    w2 = W[:, EMB_FEAT:].T
    b2 = b.reshape(1, TOKEN_DIM)

    grid = (B // BB,)
    out = pl.pallas_call(
        _fused_masked_linear,
        grid=grid,
        in_specs=[
            pl.BlockSpec((BB, N, 1, P, D), lambda i: (i, 0, 0, 0, 0)),
            pl.BlockSpec((BB, N, 1, P), lambda i: (i, 0, 0, 0)),
            pl.BlockSpec((BB, N, 1), lambda i: (i, 0, 0)),
            pl.BlockSpec((P, D, TOKEN_DIM), lambda i: (0, 0, 0)),
            pl.BlockSpec((P, TOKEN_DIM), lambda i: (0, 0)),
            pl.BlockSpec((1, TOKEN_DIM), lambda i: (0, 0)),
        ],
        out_specs=pl.BlockSpec((ROWS, TOKEN_DIM), lambda i: (i, 0)),
        out_shape=jax.ShapeDtypeStruct((M, TOKEN_DIM), jnp.float32),
    )(embs, vis, maskf, w1, w2, b2)
    return out.reshape(B, N, TOKEN_DIM)


# speedup vs baseline: 1.4834x; 1.0630x over previous
import jax
import jax.numpy as jnp
from jax.experimental import pallas as pl

B, N, T, P, D = 256, 128, 1, 7, 256
TOKEN_DIM = 128
EMB_FEAT = P * D
M = B * N

BB = 8
ROWS = BB * N


def _fused_masked_linear(x_ref, vis_ref, mask_ref, w1_ref, w2_ref, b_ref, out_ref):
    vis2d = vis_ref[:].reshape(ROWS, P)
    acc = jnp.dot(vis2d, w2_ref[:], preferred_element_type=jnp.float32)
    acc += b_ref[:]
    x4 = x_ref[:, :, 0, :, :]
    xt = jnp.transpose(x4, (2, 0, 1, 3)).astype(jnp.bfloat16)
    for p in range(P):
        acc += jnp.dot(xt[p].reshape(ROWS, D), w1_ref[p],
                       preferred_element_type=jnp.float32)
    out_ref[:] = acc * mask_ref[:].reshape(ROWS, 1)


def kernel(embs, vis, masks, W, b):
    maskf = masks.astype(jnp.float32)
    w1 = W[:, :EMB_FEAT].T.reshape(P, D, TOKEN_DIM).astype(jnp.bfloat16)
    w2 = W[:, EMB_FEAT:].T
    b2 = b.reshape(1, TOKEN_DIM)

    grid = (B // BB,)
    out = pl.pallas_call(
        _fused_masked_linear,
        grid=grid,
        in_specs=[
            pl.BlockSpec((BB, N, 1, P, D), lambda i: (i, 0, 0, 0, 0)),
            pl.BlockSpec((BB, N, 1, P), lambda i: (i, 0, 0, 0)),
            pl.BlockSpec((BB, N, 1), lambda i: (i, 0, 0)),
            pl.BlockSpec((P, D, TOKEN_DIM), lambda i: (0, 0, 0)),
            pl.BlockSpec((P, TOKEN_DIM), lambda i: (0, 0)),
            pl.BlockSpec((1, TOKEN_DIM), lambda i: (0, 0)),
        ],
        out_specs=pl.BlockSpec((ROWS, TOKEN_DIM), lambda i: (i, 0)),
        out_shape=jax.ShapeDtypeStruct((M, TOKEN_DIM), jnp.float32),
    )(embs, vis, maskf, w1, w2, b2)
    return out.reshape(B, N, TOKEN_DIM)


# speedup vs baseline: 1.5053x; 1.0148x over previous
import jax
import jax.numpy as jnp
from jax.experimental import pallas as pl

B, N, T, P, D = 256, 128, 1, 7, 256
TOKEN_DIM = 128
EMB_FEAT = P * D
M = B * N

BB = 16
ROWS = BB * N


def _fused_masked_linear(x_ref, vis_ref, mask_ref, w1_ref, w2_ref, b_ref, out_ref):
    vis2d = vis_ref[:].reshape(ROWS, P)
    acc = jnp.dot(vis2d, w2_ref[:], preferred_element_type=jnp.float32)
    acc += b_ref[:]
    x4 = x_ref[:, :, 0, :, :]
    xt = jnp.transpose(x4, (2, 0, 1, 3)).astype(jnp.bfloat16)
    for p in range(P):
        acc += jnp.dot(xt[p].reshape(ROWS, D), w1_ref[p],
                       preferred_element_type=jnp.float32)
    out_ref[:] = acc * mask_ref[:].reshape(ROWS, 1)


def kernel(embs, vis, masks, W, b):
    maskf = masks.astype(jnp.float32)
    w1 = W[:, :EMB_FEAT].T.reshape(P, D, TOKEN_DIM).astype(jnp.bfloat16)
    w2 = W[:, EMB_FEAT:].T
    b2 = b.reshape(1, TOKEN_DIM)

    grid = (B // BB,)
    out = pl.pallas_call(
        _fused_masked_linear,
        grid=grid,
        in_specs=[
            pl.BlockSpec((BB, N, 1, P, D), lambda i: (i, 0, 0, 0, 0)),
            pl.BlockSpec((BB, N, 1, P), lambda i: (i, 0, 0, 0)),
            pl.BlockSpec((BB, N, 1), lambda i: (i, 0, 0)),
            pl.BlockSpec((P, D, TOKEN_DIM), lambda i: (0, 0, 0)),
            pl.BlockSpec((P, TOKEN_DIM), lambda i: (0, 0)),
            pl.BlockSpec((1, TOKEN_DIM), lambda i: (0, 0)),
        ],
        out_specs=pl.BlockSpec((ROWS, TOKEN_DIM), lambda i: (i, 0)),
        out_shape=jax.ShapeDtypeStruct((M, TOKEN_DIM), jnp.float32),
    )(embs, vis, maskf, w1, w2, b2)
    return out.reshape(B, N, TOKEN_DIM)
